# baseline (speedup 1.0000x reference)
import functools

import jax
import jax.numpy as jnp
import numpy as np
from jax import lax
from jax.experimental import pallas as pl
from jax.experimental.pallas import tpu as pltpu
from jax.experimental.pallas import tpu_sc as plsc

_ALPHA = 4


def kernel(frames):
    B, T, C, H, W = frames.shape
    S = T // _ALPHA
    ROW = C * H * W

    idx = np.linspace(0.0, T - 1, S).astype(np.int64)
    assert np.array_equal(idx, (np.arange(S) * (T - 1)) // (S - 1))

    NW = 32
    n_out = B * S
    rows_per_w = n_out // NW

    mesh = plsc.VectorSubcoreMesh(core_axis_name="c", subcore_axis_name="s")

    @functools.partial(
        pl.kernel,
        out_type=jax.ShapeDtypeStruct((n_out * ROW,), jnp.float32),
        mesh=mesh,
        scratch_types=[pltpu.SemaphoreType.DMA],
    )
    def gather_rows(src_hbm, out_hbm, sem):
        wid = lax.axis_index("s") * 2 + lax.axis_index("c")
        copies = []
        for k in range(rows_per_w):
            r = wid * rows_per_w + k
            b = r // S
            j = r % S
            src_row = b * T + (j * (T - 1)) // (S - 1)
            copies.append(
                pltpu.async_copy(
                    src_hbm.at[pl.ds(src_row * ROW, ROW)],
                    out_hbm.at[pl.ds(r * ROW, ROW)],
                    sem,
                )
            )
        for c in copies:
            c.wait()

    slow = gather_rows(frames.reshape(-1))
    return slow.reshape(B, S, C, H, W), frames


# speedup vs baseline: 3.7723x; 3.7723x over previous
import functools

import jax
import jax.numpy as jnp
import numpy as np
from jax import lax
from jax.experimental import pallas as pl
from jax.experimental.pallas import tpu as pltpu
from jax.experimental.pallas import tpu_sc as plsc

_ALPHA = 4


def kernel(frames):
    B, T, C, H, W = frames.shape
    S = T // _ALPHA
    ROW = C * H * W

    idx = np.linspace(0.0, T - 1, S).astype(np.int64)
    assert np.array_equal(idx, (np.arange(S) * (T - 1)) // (S - 1))

    NW = 32
    n_out = B * S
    rows_per_w = n_out // NW

    mesh = plsc.VectorSubcoreMesh(core_axis_name="c", subcore_axis_name="s")

    NCHUNK = 4
    CHUNK = ROW // NCHUNK
    n_iter = rows_per_w * NCHUNK

    @functools.partial(
        pl.kernel,
        out_type=jax.ShapeDtypeStruct((n_out * ROW,), jnp.float32),
        mesh=mesh,
        scratch_types=[
            pltpu.VMEM((2, CHUNK), jnp.float32),
            pltpu.SemaphoreType.DMA,
            pltpu.SemaphoreType.DMA,
            pltpu.SemaphoreType.DMA,
            pltpu.SemaphoreType.DMA,
        ],
    )
    def gather_rows(src_hbm, out_hbm, buf, lsem0, lsem1, ssem0, ssem1):
        wid = lax.axis_index("s") * 2 + lax.axis_index("c")
        lsem = (lsem0, lsem1)
        ssem = (ssem0, ssem1)

        def offs(i):
            k, c = divmod(i, NCHUNK)
            r = wid * rows_per_w + k
            b = r // S
            j = r % S
            src_row = b * T + (j * (T - 1)) // (S - 1)
            return src_row * ROW + c * CHUNK, r * ROW + c * CHUNK

        loads = [None, None]
        stores = [None, None]

        def start_load(i):
            soff, _ = offs(i)
            loads[i % 2] = pltpu.async_copy(
                src_hbm.at[pl.ds(soff, CHUNK)], buf.at[i % 2], lsem[i % 2]
            )

        def start_store(i):
            _, doff = offs(i)
            stores[i % 2] = pltpu.async_copy(
                buf.at[i % 2], out_hbm.at[pl.ds(doff, CHUNK)], ssem[i % 2]
            )

        start_load(0)
        for i in range(n_iter):
            if i + 1 < n_iter:
                if i >= 1:
                    stores[(i + 1) % 2].wait()
                start_load(i + 1)
            loads[i % 2].wait()
            start_store(i)
        stores[(n_iter - 2) % 2].wait()
        stores[(n_iter - 1) % 2].wait()

    slow = gather_rows(frames.reshape(-1))
    return slow.reshape(B, S, C, H, W), frames
